# Initial kernel scaffold; baseline (speedup 1.0000x reference)
import jax
import jax.numpy as jnp
from jax.experimental import pallas as pl


def kernel(m, x, y):
    raise NotImplementedError("write your pallas kernel here")


# speedup vs baseline: 1.1338x; 1.1338x over previous
import functools

import jax
import jax.numpy as jnp
from jax import lax
from jax.experimental import pallas as pl
from jax.experimental.pallas import tpu as pltpu
from jax.experimental.pallas import tpu_sc as plsc

_B = 64
_P = 64
_H = 512
_W = 512
_PLANE = 2 * _H * _W

_L = 16
_NC = 2
_NS = 16
_NW = _NC * _NS
_BPW = _B // _NW
_IPW = _BPW * _P

_mesh = plsc.VectorSubcoreMesh(core_axis_name="c", subcore_axis_name="s")


@functools.partial(
    pl.kernel,
    mesh=_mesh,
    out_type=jax.ShapeDtypeStruct((_B * _P,), jnp.float32),
    scratch_types=[
        pltpu.VMEM((_P,), jnp.int32),
        pltpu.VMEM((_P,), jnp.int32),
        pltpu.VMEM((_IPW,), jnp.int32),
        pltpu.VMEM((_IPW,), jnp.float32),
        pltpu.SemaphoreType.DMA,
    ],
)
def _probe_gather(m_hbm, x_hbm, y_hbm, out_hbm, x_v, y_v, idx_v, val_v, sem):
    wid = lax.axis_index("s") * _NC + lax.axis_index("c")
    b0 = wid * _BPW
    pltpu.sync_copy(x_hbm, x_v)
    pltpu.sync_copy(y_hbm, y_v)
    for c in range(_P // _L):
        xv = x_v[pl.ds(c * _L, _L)]
        yv = y_v[pl.ds(c * _L, _L)]
        off = xv * _W + yv
        for j in range(_BPW):
            idx_v[pl.ds(j * _P + c * _L, _L)] = off + (b0 + j) * _PLANE
    pltpu.async_copy(m_hbm.at[idx_v], val_v, sem).wait()
    pltpu.sync_copy(val_v, out_hbm.at[pl.ds(b0 * _P, _IPW)])


def kernel(m, x, y):
    out = _probe_gather(
        m.reshape(-1), x.astype(jnp.int32), y.astype(jnp.int32)
    )
    return out.reshape(_B, _P)


# speedup vs baseline: 4.9759x; 4.3887x over previous
import functools

import jax
import jax.numpy as jnp
from jax import lax
from jax.experimental import pallas as pl
from jax.experimental.pallas import tpu as pltpu
from jax.experimental.pallas import tpu_sc as plsc

_B = 64
_P = 64
_H = 512
_W = 512
_ROWS_PER_B = 2 * _H

_L = 16
_NC = 2
_NS = 16
_NW = _NC * _NS
_BPW = _B // _NW
_IPW = _BPW * _P

_mesh = plsc.VectorSubcoreMesh(core_axis_name="c", subcore_axis_name="s")


@functools.partial(
    pl.kernel,
    mesh=_mesh,
    out_type=jax.ShapeDtypeStruct((_B * _P,), jnp.float32),
    scratch_types=[
        pltpu.VMEM((_P,), jnp.int32),
        pltpu.VMEM((_P,), jnp.int32),
        pltpu.VMEM((_IPW,), jnp.int32),
        pltpu.VMEM((_IPW, _W), jnp.float32),
        pltpu.VMEM((_IPW,), jnp.float32),
        pltpu.SemaphoreType.DMA,
    ],
    compiler_params=pltpu.CompilerParams(needs_layout_passes=False),
)
def _probe_gather(m_hbm, x_hbm, y_hbm, out_hbm, x_v, y_v, ridx_v, rows_v,
                  val_v, sem):
    wid = lax.axis_index("s") * _NC + lax.axis_index("c")
    b0 = wid * _BPW
    pltpu.sync_copy(x_hbm, x_v)
    pltpu.sync_copy(y_hbm, y_v)
    for c in range(_P // _L):
        xv = x_v[pl.ds(c * _L, _L)]
        for j in range(_BPW):
            ridx_v[pl.ds(j * _P + c * _L, _L)] = xv + (b0 + j) * _ROWS_PER_B
    pltpu.async_copy(m_hbm.at[ridx_v], rows_v, sem).wait()
    for c in range(_IPW // _L):
        rids = lax.iota(jnp.int32, _L) + c * _L
        yids = y_v[pl.ds((c * _L) % _P, _L)]
        val_v[pl.ds(c * _L, _L)] = plsc.load_gather(rows_v, [rids, yids])
    pltpu.sync_copy(val_v, out_hbm.at[pl.ds(b0 * _P, _IPW)])


def kernel(m, x, y):
    m2 = m.reshape(_B * 2 * _H, _W)
    out = _probe_gather(m2, x.astype(jnp.int32), y.astype(jnp.int32))
    return out.reshape(_B, _P)


# speedup vs baseline: 5.5213x; 1.1096x over previous
import functools

import jax
import jax.numpy as jnp
from jax import lax
from jax.experimental import pallas as pl
from jax.experimental.pallas import tpu as pltpu
from jax.experimental.pallas import tpu_sc as plsc

_B = 64
_P = 64
_H = 512
_W = 512

_L = 16
_NC = 2
_NS = 16
_NW = _NC * _NS
_BPW = _B // _NW
_IPW = _BPW * _P

_RUNS_PER_B = 2 * _H * _W // 128

_mesh = plsc.VectorSubcoreMesh(core_axis_name="c", subcore_axis_name="s")


@functools.partial(
    pl.kernel,
    mesh=_mesh,
    out_type=jax.ShapeDtypeStruct((_B * _P,), jnp.float32),
    scratch_types=[
        pltpu.VMEM((_P,), jnp.int32),
        pltpu.VMEM((_P,), jnp.int32),
        pltpu.VMEM((_IPW,), jnp.int32),
        pltpu.VMEM((_IPW, 128), jnp.float32),
        pltpu.VMEM((_IPW,), jnp.float32),
        pltpu.SemaphoreType.DMA,
    ],
    compiler_params=pltpu.CompilerParams(needs_layout_passes=False),
)
def _probe_gather(m_hbm, x_hbm, y_hbm, out_hbm, x_v, y_v, ridx_v, runs_v,
                  val_v, sem):
    wid = lax.axis_index("s") * _NC + lax.axis_index("c")
    b0 = wid * _BPW
    pltpu.sync_copy(x_hbm, x_v)
    pltpu.sync_copy(y_hbm, y_v)
    for c in range(_P // _L):
        xv = x_v[pl.ds(c * _L, _L)]
        yv = y_v[pl.ds(c * _L, _L)]
        run = (
            (xv >> 3) * 32
            + (yv >> 7) * 8
            + (xv & 7)
        )
        for j in range(_BPW):
            ridx_v[pl.ds(j * _P + c * _L, _L)] = run + (b0 + j) * _RUNS_PER_B
    pltpu.async_copy(m_hbm.at[ridx_v], runs_v, sem).wait()
    for c in range(_IPW // _L):
        rids = lax.iota(jnp.int32, _L) + c * _L
        lids = y_v[pl.ds((c * _L) % _P, _L)] & 127
        val_v[pl.ds(c * _L, _L)] = plsc.load_gather(runs_v, [rids, lids])
    pltpu.sync_copy(val_v, out_hbm.at[pl.ds(b0 * _P, _IPW)])


def kernel(m, x, y):
    mruns = (
        m.reshape(8192, 8, 4, 128)
        .transpose(0, 2, 1, 3)
        .reshape(_B * _RUNS_PER_B, 128)
    )
    out = _probe_gather(mruns, x.astype(jnp.int32), y.astype(jnp.int32))
    return out.reshape(_B, _P)


# speedup vs baseline: 5.6635x; 1.0258x over previous
import functools

import jax
import jax.numpy as jnp
from jax import lax
from jax.experimental import pallas as pl
from jax.experimental.pallas import tpu as pltpu
from jax.experimental.pallas import tpu_sc as plsc

_B = 64
_P = 64
_H = 512
_W = 512

_L = 16
_NC = 2
_NS = 16
_NW = _NC * _NS
_BPW = _B // _NW
_IPW = _BPW * _P

_RUNS_PER_B = 2 * _H * _W // 128

_mesh = plsc.VectorSubcoreMesh(core_axis_name="c", subcore_axis_name="s")


@functools.partial(
    pl.kernel,
    mesh=_mesh,
    out_type=jax.ShapeDtypeStruct((_B * _P,), jnp.float32),
    scratch_types=[
        pltpu.VMEM((2 * _P,), jnp.int32),
        pltpu.VMEM((_IPW,), jnp.int32),
        pltpu.VMEM((_IPW, 128), jnp.float32),
        pltpu.VMEM((_IPW,), jnp.float32),
        pltpu.SemaphoreType.DMA,
    ],
    compiler_params=pltpu.CompilerParams(needs_layout_passes=False),
)
def _probe_gather(m_hbm, xy_hbm, out_hbm, xy_v, ridx_v, runs_v, val_v, sem):
    wid = lax.axis_index("s") * _NC + lax.axis_index("c")
    b0 = wid * _BPW
    pltpu.sync_copy(xy_hbm, xy_v)
    for c in range(_P // _L):
        xv = xy_v[pl.ds(c * _L, _L)]
        yv = xy_v[pl.ds(_P + c * _L, _L)]
        run = (
            (xv >> 3) * 32
            + (yv >> 7) * 8
            + (xv & 7)
        )
        for j in range(_BPW):
            ridx_v[pl.ds(j * _P + c * _L, _L)] = run + (b0 + j) * _RUNS_PER_B
    pltpu.async_copy(m_hbm.at[ridx_v], runs_v, sem).wait()
    for c in range(_IPW // _L):
        rids = lax.iota(jnp.int32, _L) + c * _L
        lids = xy_v[pl.ds(_P + (c * _L) % _P, _L)] & 127
        val_v[pl.ds(c * _L, _L)] = plsc.load_gather(runs_v, [rids, lids])
    pltpu.sync_copy(val_v, out_hbm.at[pl.ds(b0 * _P, _IPW)])


def kernel(m, x, y):
    mruns = (
        m.reshape(8192, 8, 4, 128)
        .transpose(0, 2, 1, 3)
        .reshape(_B * _RUNS_PER_B, 128)
    )
    xy = jnp.concatenate([x.astype(jnp.int32), y.astype(jnp.int32)])
    out = _probe_gather(mruns, xy)
    return out.reshape(_B, _P)


# speedup vs baseline: 6.0388x; 1.0663x over previous
import functools

import jax
import jax.numpy as jnp
from jax import lax
from jax.experimental import pallas as pl
from jax.experimental.pallas import tpu as pltpu
from jax.experimental.pallas import tpu_sc as plsc

_B = 64
_P = 64
_H = 512
_W = 512

_L = 16
_NC = 2
_NS = 16
_NW = _NC * _NS
_BPW = _B // _NW
_IPW = _BPW * _P

_RUNS_PER_B = 2 * _H * _W // 128

_mesh = plsc.VectorSubcoreMesh(core_axis_name="c", subcore_axis_name="s")


@functools.partial(
    pl.kernel,
    mesh=_mesh,
    out_type=jax.ShapeDtypeStruct((_B, _P), jnp.float32),
    scratch_types=[
        pltpu.VMEM((2 * _P,), jnp.int32),
        pltpu.VMEM((_IPW,), jnp.int32),
        pltpu.VMEM((_IPW, 128), jnp.float32),
        pltpu.VMEM((_BPW, _P), jnp.float32),
        pltpu.SemaphoreType.DMA,
    ],
    compiler_params=pltpu.CompilerParams(needs_layout_passes=False),
)
def _probe_gather(m_hbm, xy_hbm, out_hbm, xy_v, ridx_v, runs_v, val_v, sem):
    wid = lax.axis_index("s") * _NC + lax.axis_index("c")
    b0 = wid * _BPW
    pltpu.sync_copy(xy_hbm, xy_v)

    def compute_idx(c, carry):
        xv = xy_v[pl.ds(c * _L, _L)]
        yv = xy_v[pl.ds(_P + c * _L, _L)]
        run = (xv >> 3) * 32 + (yv >> 7) * 8 + (xv & 7)
        for j in range(_BPW):
            ridx_v[pl.ds(j * _P + c * _L, _L)] = run + (b0 + j) * _RUNS_PER_B
        return carry

    lax.fori_loop(0, _P // _L, compute_idx, 0, unroll=False)
    pltpu.async_copy(m_hbm.at[ridx_v], runs_v, sem).wait()

    def extract(c, carry):
        rids = lax.iota(jnp.int32, _L) + c * _L
        lids = xy_v[pl.ds(_P + (c & 3) * _L, _L)] & 127
        vals = plsc.load_gather(runs_v, [rids, lids])
        val_v[c >> 2, pl.ds((c & 3) * _L, _L)] = vals
        return carry

    lax.fori_loop(0, _IPW // _L, extract, 0, unroll=False)
    pltpu.sync_copy(val_v, out_hbm.at[pl.ds(b0, _BPW)])


def kernel(m, x, y):
    mruns = (
        m.reshape(8192, 8, 4, 128)
        .transpose(0, 2, 1, 3)
        .reshape(_B * _RUNS_PER_B, 128)
    )
    xy = jnp.concatenate([x.astype(jnp.int32), y.astype(jnp.int32)])
    return _probe_gather(mruns, xy)
